# baseline (speedup 1.0000x reference)
import jax
import jax.numpy as jnp
from jax.experimental import pallas as pl
from jax.experimental.pallas import tpu as pltpu

_BH = 200


def _fused_body(x_ref, w_ref, adj_a_ref, adj_b_ref, b_ref, out_ref):
    bh = adj_a_ref.shape[0]
    ta = jnp.dot(adj_a_ref[...], x_ref[...],
                 preferred_element_type=jnp.float32)
    out_ref[pl.ds(0, bh), :] = jnp.dot(
        ta, w_ref[...], preferred_element_type=jnp.float32) + b_ref[...]
    tb = jnp.dot(adj_b_ref[...], x_ref[...],
                 preferred_element_type=jnp.float32)
    out_ref[pl.ds(bh, bh), :] = jnp.dot(
        tb, w_ref[...], preferred_element_type=jnp.float32) + b_ref[...]


def kernel(input, adj, W, b):
    n, d_in = input.shape
    d_out = W.shape[1]
    b2 = b.reshape(1, d_out)
    num_i = n // (2 * _BH)
    out = pl.pallas_call(
        _fused_body,
        grid=(num_i,),
        in_specs=[
            pl.BlockSpec((n, d_in), lambda i: (0, 0)),
            pl.BlockSpec((d_in, d_out), lambda i: (0, 0)),
            pl.BlockSpec((_BH, n), lambda i: (2 * i, 0)),
            pl.BlockSpec((_BH, n), lambda i: (2 * i + 1, 0)),
            pl.BlockSpec((1, d_out), lambda i: (0, 0)),
        ],
        out_specs=pl.BlockSpec((2 * _BH, d_out), lambda i: (i, 0)),
        out_shape=jax.ShapeDtypeStruct((n, d_out), jnp.float32),
        compiler_params=pltpu.CompilerParams(
            dimension_semantics=("arbitrary",)),
    )(input, W, adj, adj, b2)
    return out


# speedup vs baseline: 1.0904x; 1.0904x over previous
import jax
import jax.numpy as jnp
from jax.experimental import pallas as pl
from jax.experimental.pallas import tpu as pltpu

_BI = 400


def _fused_body(x_ref, w_ref, adj_ref, b_ref, out_ref, sup_ref):
    @pl.when(pl.program_id(0) == 0)
    def _():
        sup_ref[...] = jnp.dot(x_ref[...], w_ref[...],
                               preferred_element_type=jnp.float32)

    out_ref[...] = jnp.dot(adj_ref[...], sup_ref[...],
                           preferred_element_type=jnp.float32) + b_ref[...]


def kernel(input, adj, W, b):
    n, d_in = input.shape
    d_out = W.shape[1]
    b2 = b.reshape(1, d_out)
    num_i = n // _BI
    out = pl.pallas_call(
        _fused_body,
        grid=(num_i,),
        in_specs=[
            pl.BlockSpec((n, d_in), lambda i: (0, 0)),
            pl.BlockSpec((d_in, d_out), lambda i: (0, 0)),
            pl.BlockSpec((_BI, n), lambda i: (i, 0)),
            pl.BlockSpec((1, d_out), lambda i: (0, 0)),
        ],
        out_specs=pl.BlockSpec((_BI, d_out), lambda i: (i, 0)),
        out_shape=jax.ShapeDtypeStruct((n, d_out), jnp.float32),
        scratch_shapes=[pltpu.VMEM((n, d_out), jnp.float32)],
        compiler_params=pltpu.CompilerParams(
            dimension_semantics=("arbitrary",)),
    )(input, W, adj, b2)
    return out
